# baseline (speedup 1.0000x reference)
import functools

import jax
import jax.numpy as jnp
from jax import lax
from jax.experimental import pallas as pl
from jax.experimental.pallas import tpu as pltpu
from jax.experimental.pallas import tpu_sc as plsc

_FIELD_DIMS = (64, 32, 32, 16, 16, 32, 32, 64, 32, 32)
_NUM_FIELDS = len(_FIELD_DIMS)
_OFFSETS = tuple(sum(_FIELD_DIMS[:i]) for i in range(_NUM_FIELDS))
_CW = sum(_FIELD_DIMS)
_B, _S, _D = 4096, 50, 256
_N = _B * _S

_BIG = (0, 7, 8, 9)
_WIN = (0, 64, 0, 32)
_SMALL = (1, 2, 3, 4, 5, 6)
_NBIG = len(_BIG)
_NSMALL = len(_SMALL)
_SEG = 128
_KCAT = (_NBIG + _NSMALL) * _SEG

_NCHUNK = 4
_NTOK = _N // _NCHUNK
_BCH = _B // _NCHUNK

_INFO = plsc.get_sparse_core_info()
_NC, _NS = _INFO.num_cores, _INFO.num_subcores
_NW = _NC * _NS
_G = 128
_NGRPT = _NTOK // _G


def _sc_gather_body(ids_hbm, *rest):
    tables = rest[:_NBIG]
    outs = rest[_NBIG:2 * _NBIG]
    idx_v = rest[2 * _NBIG]
    rbufs = rest[2 * _NBIG + 1:3 * _NBIG + 1]
    sem_g, sem_w, sem_i = rest[-3], rest[-2], rest[-1]

    wid = lax.axis_index("s") * _NC + lax.axis_index("c")
    ngroups = jnp.where(wid < 16, 13, 12)
    gbase = wid * 12 + jnp.minimum(wid, 16)

    pltpu.make_async_copy(
        ids_hbm.at[:, pl.ds(gbase * _G, _G)], idx_v, sem_i).start()

    def body(g, carry):
        pltpu.make_async_copy(
            ids_hbm.at[:, pl.ds(gbase * _G, _G)], idx_v, sem_i).wait()
        gathers = []
        for f in range(_NBIG):
            cp = pltpu.make_async_copy(tables[f].at[idx_v.at[f]], rbufs[f],
                                       sem_g)
            cp.start()
            gathers.append(cp)
        base = (gbase + g) * _G
        writes = []
        for f in range(_NBIG):
            gathers[f].wait()
            cp = pltpu.make_async_copy(
                rbufs[f], outs[f].at[pl.ds(base, _G), :], sem_w)
            cp.start()
            writes.append(cp)
        nbase = jnp.minimum(gbase + g + 1, _NGRPT - 1) * _G
        pltpu.make_async_copy(
            ids_hbm.at[:, pl.ds(nbase, _G)], idx_v, sem_i).start()
        for cp in writes:
            cp.wait()
        return carry

    lax.fori_loop(0, ngroups, body, 0)
    pltpu.make_async_copy(
        ids_hbm.at[:, pl.ds(0, _G)], idx_v, sem_i).wait()


_sc_gather = functools.partial(
    pl.kernel,
    mesh=plsc.VectorSubcoreMesh(core_axis_name="c", subcore_axis_name="s"),
    out_type=tuple(jax.ShapeDtypeStruct((_NTOK, _SEG), jnp.float32)
                   for _ in range(_NBIG)),
    scratch_types=(
        [pltpu.VMEM((_NBIG, _G), jnp.int32)]
        + [pltpu.VMEM((_G, _SEG), jnp.float32) for _ in range(_NBIG)]
        + [pltpu.SemaphoreType.DMA] * 3
    ),
)(_sc_gather_body)


def _pstack_body(t_ref, w_ref, o_ref):
    o_ref[...] = jnp.dot(t_ref[...], w_ref[...],
                         preferred_element_type=jnp.float32).astype(jnp.bfloat16)


def _pstack(t_pad, w):
    return pl.pallas_call(
        _pstack_body,
        out_shape=jax.ShapeDtypeStruct((_NSMALL * _SEG, _D), jnp.bfloat16),
    )(t_pad, w)


_BN = 1600
_BB = _BN // _S


def _tc_body(*refs):
    nskip = len(refs) - (_NBIG + 6)
    g_refs = refs[nskip:nskip + _NBIG]
    ids_ref, w_ref, b_ref, ga_ref, be_ref, o_ref = refs[nskip + _NBIG:]
    parts = [r[...].astype(jnp.bfloat16) for r in g_refs]
    iota = lax.broadcasted_iota(jnp.int32, (_BN, _SEG), 1)
    for j in range(_NSMALL):
        ids_j = ids_ref[:, j]
        parts.append((iota == ids_j[:, None]).astype(jnp.bfloat16))
    x = jnp.concatenate(parts, axis=1)
    y = jnp.dot(x, w_ref[...], preferred_element_type=jnp.float32) + b_ref[...]
    mean = jnp.mean(y, axis=1, keepdims=True)
    yc = y - mean
    var = jnp.mean(yc * yc, axis=1, keepdims=True)
    out = yc * lax.rsqrt(var + 1e-5) * ga_ref[...] + be_ref[...]
    o_ref[...] = out.reshape(_BB, _S, _D)


def _tc_project(chunk, acc, fields, ids_small, wcat, b, gamma, beta):
    base = chunk * (_BCH // _BB)
    acc_args = () if acc is None else (acc,)
    acc_specs = [] if acc is None else [pl.BlockSpec(memory_space=pl.ANY)]
    return pl.pallas_call(
        _tc_body,
        grid=(_NTOK // _BN,),
        in_specs=(
            acc_specs
            + [pl.BlockSpec((_BN, _SEG), lambda i: (i, 0)) for _ in range(_NBIG)]
            + [
                pl.BlockSpec((_BN, _NSMALL), lambda i: (i, 0)),
                pl.BlockSpec((_KCAT, _D), lambda i: (0, 0)),
                pl.BlockSpec((1, _D), lambda i: (0, 0)),
                pl.BlockSpec((1, _D), lambda i: (0, 0)),
                pl.BlockSpec((1, _D), lambda i: (0, 0)),
            ]
        ),
        out_specs=pl.BlockSpec((_BB, _S, _D), lambda i: (base + i, 0, 0)),
        out_shape=jax.ShapeDtypeStruct((_B, _S, _D), jnp.float32),
        input_output_aliases={} if acc is None else {0: 0},
        compiler_params=pltpu.CompilerParams(
            dimension_semantics=("arbitrary",)),
    )(*acc_args, *fields, ids_small, wcat, b, gamma, beta)


def kernel(input_ids_surface, input_ids_pos, input_ids_pos_detail1,
           input_ids_pos_detail2, input_ids_pos_detail3,
           input_ids_conjugated_type, input_ids_conjugated_form,
           input_ids_lemma, input_ids_base_orth, input_ids_reading,
           table_surface, table_pos, table_pos_detail1,
           table_pos_detail2, table_pos_detail3,
           table_conjugated_type, table_conjugated_form,
           table_lemma, table_base_orth, table_reading,
           W, b, gamma, beta):
    ids = [input_ids_surface, input_ids_pos, input_ids_pos_detail1,
           input_ids_pos_detail2, input_ids_pos_detail3,
           input_ids_conjugated_type, input_ids_conjugated_form,
           input_ids_lemma, input_ids_base_orth, input_ids_reading]
    tables = [table_surface, table_pos, table_pos_detail1,
              table_pos_detail2, table_pos_detail3,
              table_conjugated_type, table_conjugated_form,
              table_lemma, table_base_orth, table_reading]

    ids_big = jnp.stack([ids[f].reshape(-1) for f in _BIG])
    t1 = jnp.concatenate([tables[0], tables[7]], axis=1)
    t2 = jnp.pad(jnp.concatenate([tables[8], tables[9]], axis=1),
                 ((0, 0), (0, 64)))
    tables_big = [t1, t1, t2, t2]

    wseg = []
    for f, win in zip(_BIG, _WIN):
        wf = W[_OFFSETS[f]:_OFFSETS[f] + _FIELD_DIMS[f], :]
        wseg.append(jnp.pad(wf, ((win, _SEG - win - _FIELD_DIMS[f]), (0, 0))))
    w_big = jnp.concatenate(wseg, axis=0).astype(jnp.bfloat16)

    t_pad = jnp.zeros((_NSMALL * _SEG, _CW), jnp.float32)
    for k, f in enumerate(_SMALL):
        t_pad = t_pad.at[k * _SEG:k * _SEG + tables[f].shape[0],
                         _OFFSETS[f]:_OFFSETS[f] + _FIELD_DIMS[f]].set(tables[f])
    p_small = _pstack(t_pad, W)
    wcat = jnp.concatenate([w_big, p_small], axis=0)

    ids_small = jnp.stack([ids[f].reshape(-1) for f in _SMALL], axis=-1)
    b2, ga2, be2 = b.reshape(1, _D), gamma.reshape(1, _D), beta.reshape(1, _D)

    gathered = []
    for c in range(_NCHUNK):
        sl = slice(c * _NTOK, (c + 1) * _NTOK)
        gathered.append(_sc_gather(ids_big[:, sl], *tables_big))
    acc = None
    for c in range(_NCHUNK):
        sl = slice(c * _NTOK, (c + 1) * _NTOK)
        acc = _tc_project(c, acc, gathered[c], ids_small[sl],
                          wcat, b2, ga2, be2)
    return acc


# speedup vs baseline: 1.0381x; 1.0381x over previous
import functools

import jax
import jax.numpy as jnp
from jax import lax
from jax.experimental import pallas as pl
from jax.experimental.pallas import tpu as pltpu
from jax.experimental.pallas import tpu_sc as plsc

_FIELD_DIMS = (64, 32, 32, 16, 16, 32, 32, 64, 32, 32)
_NUM_FIELDS = len(_FIELD_DIMS)
_OFFSETS = tuple(sum(_FIELD_DIMS[:i]) for i in range(_NUM_FIELDS))
_CW = sum(_FIELD_DIMS)
_B, _S, _D = 4096, 50, 256
_N = _B * _S

_BIG = (0, 7, 8, 9)
_WIN = (0, 64, 0, 32)
_SMALL = (1, 2, 3, 4, 5, 6)
_NBIG = len(_BIG)
_NSMALL = len(_SMALL)
_SEG = 128
_KCAT = (_NBIG + _NSMALL) * _SEG

_NCHUNK = 4
_NTOK = _N // _NCHUNK
_BCH = _B // _NCHUNK

_INFO = plsc.get_sparse_core_info()
_NC, _NS = _INFO.num_cores, _INFO.num_subcores
_NW = _NC * _NS
_G = 128


def _sc_gather_body(ids_hbm, *rest):
    tables = rest[:_NBIG]
    outs = rest[_NBIG:2 * _NBIG]
    idx_v = rest[2 * _NBIG]
    rbufs = rest[2 * _NBIG + 1:3 * _NBIG + 1]
    sem_g, sem_w = rest[-2], rest[-1]

    wid = lax.axis_index("s") * _NC + lax.axis_index("c")
    ngroups = jnp.where(wid < 16, 13, 12)
    gbase = wid * 12 + jnp.minimum(wid, 16)

    def body(g, carry):
        base = (gbase + g) * _G
        pltpu.sync_copy(ids_hbm.at[:, pl.ds(base, _G)], idx_v)
        gathers = []
        for f in range(_NBIG):
            cp = pltpu.make_async_copy(tables[f].at[idx_v.at[f]], rbufs[f],
                                       sem_g)
            cp.start()
            gathers.append(cp)
        for cp in gathers:
            cp.wait()
        writes = []
        for f in range(_NBIG):
            cp = pltpu.make_async_copy(
                rbufs[f], outs[f].at[pl.ds(base, _G), :], sem_w)
            cp.start()
            writes.append(cp)
        for cp in writes:
            cp.wait()
        return carry

    lax.fori_loop(0, ngroups, body, 0)


_sc_gather = functools.partial(
    pl.kernel,
    mesh=plsc.VectorSubcoreMesh(core_axis_name="c", subcore_axis_name="s"),
    out_type=tuple(jax.ShapeDtypeStruct((_NTOK, _SEG), jnp.float32)
                   for _ in range(_NBIG)),
    scratch_types=(
        [pltpu.VMEM((_NBIG, _G), jnp.int32)]
        + [pltpu.VMEM((_G, _SEG), jnp.float32) for _ in range(_NBIG)]
        + [pltpu.SemaphoreType.DMA, pltpu.SemaphoreType.DMA]
    ),
)(_sc_gather_body)


def _pstack_body(t_ref, w_ref, o_ref):
    o_ref[...] = jnp.dot(t_ref[...], w_ref[...],
                         preferred_element_type=jnp.float32).astype(jnp.bfloat16)


def _pstack(t_pad, w):
    return pl.pallas_call(
        _pstack_body,
        out_shape=jax.ShapeDtypeStruct((_NSMALL * _SEG, _D), jnp.bfloat16),
    )(t_pad, w)


_BN = 3200
_BB = _BN // _S


def _tc_body(*refs):
    nskip = len(refs) - (_NBIG + 6)
    g_refs = refs[nskip:nskip + _NBIG]
    ids_ref, w_ref, b_ref, ga_ref, be_ref, o_ref = refs[nskip + _NBIG:]
    parts = [r[...].astype(jnp.bfloat16) for r in g_refs]
    iota = lax.broadcasted_iota(jnp.int32, (_BN, _SEG), 1)
    for j in range(_NSMALL):
        ids_j = ids_ref[:, j]
        parts.append((iota == ids_j[:, None]).astype(jnp.bfloat16))
    x = jnp.concatenate(parts, axis=1)
    y = jnp.dot(x, w_ref[...], preferred_element_type=jnp.float32) + b_ref[...]
    mean = jnp.mean(y, axis=1, keepdims=True)
    yc = y - mean
    var = jnp.mean(yc * yc, axis=1, keepdims=True)
    out = yc * lax.rsqrt(var + 1e-5) * ga_ref[...] + be_ref[...]
    o_ref[...] = out.reshape(_BB, _S, _D)


def _tc_project(chunk, acc, fields, ids_small, wcat, b, gamma, beta):
    base = chunk * (_BCH // _BB)
    acc_args = () if acc is None else (acc,)
    acc_specs = [] if acc is None else [pl.BlockSpec(memory_space=pl.ANY)]
    return pl.pallas_call(
        _tc_body,
        grid=(_NTOK // _BN,),
        in_specs=(
            acc_specs
            + [pl.BlockSpec((_BN, _SEG), lambda i: (i, 0)) for _ in range(_NBIG)]
            + [
                pl.BlockSpec((_BN, _NSMALL), lambda i: (i, 0)),
                pl.BlockSpec((_KCAT, _D), lambda i: (0, 0)),
                pl.BlockSpec((1, _D), lambda i: (0, 0)),
                pl.BlockSpec((1, _D), lambda i: (0, 0)),
                pl.BlockSpec((1, _D), lambda i: (0, 0)),
            ]
        ),
        out_specs=pl.BlockSpec((_BB, _S, _D), lambda i: (base + i, 0, 0)),
        out_shape=jax.ShapeDtypeStruct((_B, _S, _D), jnp.float32),
        input_output_aliases={} if acc is None else {0: 0},
        compiler_params=pltpu.CompilerParams(
            dimension_semantics=("arbitrary",)),
    )(*acc_args, *fields, ids_small, wcat, b, gamma, beta)


def kernel(input_ids_surface, input_ids_pos, input_ids_pos_detail1,
           input_ids_pos_detail2, input_ids_pos_detail3,
           input_ids_conjugated_type, input_ids_conjugated_form,
           input_ids_lemma, input_ids_base_orth, input_ids_reading,
           table_surface, table_pos, table_pos_detail1,
           table_pos_detail2, table_pos_detail3,
           table_conjugated_type, table_conjugated_form,
           table_lemma, table_base_orth, table_reading,
           W, b, gamma, beta):
    ids = [input_ids_surface, input_ids_pos, input_ids_pos_detail1,
           input_ids_pos_detail2, input_ids_pos_detail3,
           input_ids_conjugated_type, input_ids_conjugated_form,
           input_ids_lemma, input_ids_base_orth, input_ids_reading]
    tables = [table_surface, table_pos, table_pos_detail1,
              table_pos_detail2, table_pos_detail3,
              table_conjugated_type, table_conjugated_form,
              table_lemma, table_base_orth, table_reading]

    ids_big = jnp.stack([ids[f].reshape(-1) for f in _BIG])
    t1 = jnp.concatenate([tables[0], tables[7]], axis=1)
    t2 = jnp.pad(jnp.concatenate([tables[8], tables[9]], axis=1),
                 ((0, 0), (0, 64)))
    tables_big = [t1, t1, t2, t2]

    wseg = []
    for f, win in zip(_BIG, _WIN):
        wf = W[_OFFSETS[f]:_OFFSETS[f] + _FIELD_DIMS[f], :]
        wseg.append(jnp.pad(wf, ((win, _SEG - win - _FIELD_DIMS[f]), (0, 0))))
    w_big = jnp.concatenate(wseg, axis=0).astype(jnp.bfloat16)

    t_pad = jnp.zeros((_NSMALL * _SEG, _CW), jnp.float32)
    for k, f in enumerate(_SMALL):
        t_pad = t_pad.at[k * _SEG:k * _SEG + tables[f].shape[0],
                         _OFFSETS[f]:_OFFSETS[f] + _FIELD_DIMS[f]].set(tables[f])
    p_small = _pstack(t_pad, W)
    wcat = jnp.concatenate([w_big, p_small], axis=0)

    ids_small = jnp.stack([ids[f].reshape(-1) for f in _SMALL], axis=-1)
    b2, ga2, be2 = b.reshape(1, _D), gamma.reshape(1, _D), beta.reshape(1, _D)

    gathered = []
    for c in range(_NCHUNK):
        sl = slice(c * _NTOK, (c + 1) * _NTOK)
        gathered.append(_sc_gather(ids_big[:, sl], *tables_big))
    acc = None
    for c in range(_NCHUNK):
        sl = slice(c * _NTOK, (c + 1) * _NTOK)
        acc = _tc_project(c, acc, gathered[c], ids_small[sl],
                          wcat, b2, ga2, be2)
    return acc
